# baseline (speedup 1.0000x reference)
import functools

import jax
import jax.numpy as jnp
from jax import lax
from jax.experimental import pallas as pl
from jax.experimental.pallas import tpu as pltpu
from jax.experimental.pallas import tpu_sc as plsc

B = 16384
D = 32
L = 16

_info = plsc.get_sparse_core_info()
NC = _info.num_cores
NS = _info.num_subcores
NW = NC * NS
B_PER_W = B // NW
GCHUNK = 128
NCHUNKS = B_PER_W // GCHUNK


def _body(uid_hbm, mid_hbm, ut_hbm, mt_hbm, out_hbm,
          idxu_v, idxm_v, urows_v, mrows_v, hb_v, outv_v, sem_u, sem_m):
    wid = lax.axis_index("s") * NC + lax.axis_index("c")
    base = wid * B_PER_W

    pltpu.sync_copy(uid_hbm.at[pl.ds(base, B_PER_W)], idxu_v)
    pltpu.sync_copy(mid_hbm.at[pl.ds(base, B_PER_W)], idxm_v)

    copies = []
    for c in range(NCHUNKS):
        s = c * GCHUNK
        copies.append(pltpu.async_copy(
            ut_hbm.at[idxu_v.at[pl.ds(s, GCHUNK)]],
            urows_v.at[pl.ds(s, GCHUNK), :], sem_u))
        copies.append(pltpu.async_copy(
            mt_hbm.at[idxm_v.at[pl.ds(s, GCHUNK)]],
            mrows_v.at[pl.ds(s, GCHUNK), :], sem_m))
    for cp in copies:
        cp.wait()

    lanes16 = lax.iota(jnp.int32, 16) * 16

    def block(b, _):
        rbase = b * 16
        for r in range(16):
            row = rbase + r
            p = (urows_v[row, pl.ds(0, 16)] * mrows_v[row, pl.ds(0, 16)]
                 + urows_v[row, pl.ds(16, 16)] * mrows_v[row, pl.ds(16, 16)])
            hb_v[pl.ds(r * 16, 16)] = p
        acc = jnp.zeros((16,), jnp.float32)
        for j in range(16):
            acc = acc + plsc.load_gather(hb_v, [lanes16 + j])
        outv_v[pl.ds(rbase, 16)] = acc
        return 0

    lax.fori_loop(0, B_PER_W // 16, block, 0)

    pltpu.sync_copy(outv_v, out_hbm.at[pl.ds(base, B_PER_W)])


@jax.jit
def _run(user_ids, movie_ids, user_table, movie_table):
    mesh = plsc.VectorSubcoreMesh(core_axis_name="c", subcore_axis_name="s")
    k = pl.kernel(
        _body,
        mesh=mesh,
        out_type=jax.ShapeDtypeStruct((B,), jnp.float32),
        scratch_types=[
            pltpu.VMEM((B_PER_W,), jnp.int32),
            pltpu.VMEM((B_PER_W,), jnp.int32),
            pltpu.VMEM((B_PER_W, D), jnp.float32),
            pltpu.VMEM((B_PER_W, D), jnp.float32),
            pltpu.VMEM((16 * 16,), jnp.float32),
            pltpu.VMEM((B_PER_W,), jnp.float32),
            pltpu.SemaphoreType.DMA,
            pltpu.SemaphoreType.DMA,
        ],
        compiler_params=pltpu.CompilerParams(
            needs_layout_passes=False, use_tc_tiling_on_sc=False),
    )
    return k(user_ids, movie_ids, user_table, movie_table)


def kernel(user_ids, movie_ids, user_table, movie_table):
    return _run(user_ids, movie_ids, user_table, movie_table)
